# baseline (speedup 1.0000x reference)
import jax
import jax.numpy as jnp
from jax.experimental import pallas as pl
from jax.experimental.pallas import tpu as pltpu

B, C, H, W = 4, 256, 224, 224
K = 64
HB = 56
NH = H // HB


def _score_body(x_ref, out_ref):
    part = jnp.sum(jnp.abs(x_ref[...]), axis=(1, 2))

    @pl.when(pl.program_id(1) == 0)
    def _init():
        out_ref[0] = part

    @pl.when(pl.program_id(1) > 0)
    def _acc():
        out_ref[0] += part


def _rank_apply_body(scol_ref, srow_ref, x_ref, out_ref, mask_scr):
    h = pl.program_id(1)

    @pl.when(h == 0)
    def _rank():
        sc = jnp.broadcast_to(scol_ref[...], (C, C))
        sr = jnp.broadcast_to(srow_ref[0], (C, C))
        ii = jax.lax.broadcasted_iota(jnp.int32, (C, C), 0)
        jj = jax.lax.broadcasted_iota(jnp.int32, (C, C), 1)
        beats_t = (sc > sr) | ((sc == sr) & (ii < jj))
        rank_row = jnp.sum(beats_t.astype(jnp.int32), axis=0, keepdims=True)
        mask_scr[...] = jnp.where(rank_row < K, 1.0, 0.0)

    @pl.when(h > 0)
    def _apply():
        out_ref[...] = x_ref[...] * mask_scr[...]


def kernel(x):
    xt = jnp.transpose(x, (0, 2, 3, 1))

    scores = pl.pallas_call(
        _score_body,
        grid=(B, NH),
        in_specs=[pl.BlockSpec((1, HB, W, C), lambda b, h: (b, h, 0, 0))],
        out_specs=pl.BlockSpec((1, 1, C), lambda b, h: (b, 0, 0)),
        out_shape=jax.ShapeDtypeStruct((B, 1, C), jnp.float32),
    )(xt)

    out_t = pl.pallas_call(
        _rank_apply_body,
        grid=(B, NH + 1),
        in_specs=[
            pl.BlockSpec((C, 1), lambda b, h: (b, 0)),
            pl.BlockSpec((1, 1, C), lambda b, h: (b, 0, 0)),
            pl.BlockSpec(
                (1, HB, W, C),
                lambda b, h: (b, jnp.maximum(h - 1, 0), 0, 0),
            ),
        ],
        out_specs=pl.BlockSpec(
            (1, HB, W, C), lambda b, h: (b, jnp.maximum(h - 1, 0), 0, 0)
        ),
        out_shape=jax.ShapeDtypeStruct((B, H, W, C), jnp.float32),
        scratch_shapes=[pltpu.VMEM((1, C), jnp.float32)],
    )(scores.reshape(B * C, 1), scores, xt)

    return jnp.transpose(out_t, (0, 3, 1, 2))
